# baseline (speedup 1.0000x reference)
import functools

import jax
import jax.numpy as jnp
from jax import lax
from jax.experimental import pallas as pl
from jax.experimental.pallas import tpu as pltpu
from jax.experimental.pallas import tpu_sc as plsc

N_Q = 100000
N_L = 1000
D = 64
T = 1024 * 200

NC = 2
NS = 16
LANES = 16
NW = NC * NS
NSLICE = 5
TS = T // NSLICE
TPW = TS // NW
CHUNK = 128
NPAIR = TPW // (2 * CHUNK)


def _sc_body(cid_hbm, qbun_hbm, qmeta_hbm, qid_t, qbid_t, lid_t,
             qid_o, qbid_o, lid_o, qm_o,
             cid_v, qidx_v, lidx_v, bund_v, qid_v, qbid_v, lid_v, qm_v,
             semg_a, semg_b, semw_a, semw_b):
    wid = lax.axis_index("s") * NC + lax.axis_index("c")
    wbase = wid * TPW
    semg = (semg_a, semg_b)
    semw = (semw_a, semw_b)

    def load_idx(h, base):
        pltpu.sync_copy(cid_hbm.at[pl.ds(base, CHUNK)], cid_v.at[h])
        for j in range(CHUNK // LANES):
            sl = pl.ds(j * LANES, LANES)
            cv = cid_v[h, sl]
            qidx_v[h, sl] = jnp.minimum(cv, N_Q - 1)
            lidx_v[h, sl] = jnp.maximum(cv - N_Q, 0)

    def fire_gathers(h):
        qix = qidx_v.at[h]
        lix = lidx_v.at[h]
        return [
            pltpu.async_copy(qbun_hbm.at[qix], bund_v.at[h], semg[h]),
            pltpu.async_copy(qmeta_hbm.at[qix], qm_v.at[h], semg[h]),
            pltpu.async_copy(qid_t.at[qix], qid_v.at[h], semg[h]),
            pltpu.async_copy(lid_t.at[lix], lid_v.at[h], semg[h]),
        ]

    def pair_body(p, carry):
        base_a = wbase + p * (2 * CHUNK)
        base_b = base_a + CHUNK
        load_idx(0, base_a)
        g_a = fire_gathers(0)
        load_idx(1, base_b)
        g_b = fire_gathers(1)
        for d in g_a:
            d.wait()
        h2_a = pltpu.async_copy(qbid_t.at[bund_v.at[0]], qbid_v.at[0], semg[0])
        w_a = [
            pltpu.async_copy(qid_v.at[0], qid_o.at[pl.ds(base_a, CHUNK)], semw[0]),
            pltpu.async_copy(lid_v.at[0], lid_o.at[pl.ds(base_a, CHUNK)], semw[0]),
            pltpu.async_copy(qm_v.at[0], qm_o.at[pl.ds(base_a, CHUNK)], semw[0]),
        ]
        for d in g_b:
            d.wait()
        h2_b = pltpu.async_copy(qbid_t.at[bund_v.at[1]], qbid_v.at[1], semg[1])
        w_b = [
            pltpu.async_copy(qid_v.at[1], qid_o.at[pl.ds(base_b, CHUNK)], semw[1]),
            pltpu.async_copy(lid_v.at[1], lid_o.at[pl.ds(base_b, CHUNK)], semw[1]),
            pltpu.async_copy(qm_v.at[1], qm_o.at[pl.ds(base_b, CHUNK)], semw[1]),
        ]
        h2_a.wait()
        w_a.append(pltpu.async_copy(qbid_v.at[0], qbid_o.at[pl.ds(base_a, CHUNK)], semw[0]))
        h2_b.wait()
        w_b.append(pltpu.async_copy(qbid_v.at[1], qbid_o.at[pl.ds(base_b, CHUNK)], semw[1]))
        for d in w_a:
            d.wait()
        for d in w_b:
            d.wait()
        return carry

    lax.fori_loop(0, NPAIR, pair_body, 0)


@functools.lru_cache(maxsize=1)
def _get_sc_gather():
    return functools.partial(
        pl.kernel,
        mesh=plsc.VectorSubcoreMesh(core_axis_name="c", subcore_axis_name="s"),
        compiler_params=pltpu.CompilerParams(use_tc_tiling_on_sc=False),
        out_type=[
            jax.ShapeDtypeStruct((TS, D), jnp.float32),
            jax.ShapeDtypeStruct((TS, D), jnp.float32),
            jax.ShapeDtypeStruct((TS, D), jnp.float32),
            jax.ShapeDtypeStruct((TS, 8), jnp.int32),
        ],
        scratch_types=[
            pltpu.VMEM((2, CHUNK), jnp.int32),
            pltpu.VMEM((2, CHUNK), jnp.int32),
            pltpu.VMEM((2, CHUNK), jnp.int32),
            pltpu.VMEM((2, CHUNK), jnp.int32),
            pltpu.VMEM((2, CHUNK, D), jnp.float32),
            pltpu.VMEM((2, CHUNK, D), jnp.float32),
            pltpu.VMEM((2, CHUNK, D), jnp.float32),
            pltpu.VMEM((2, CHUNK, 8), jnp.int32),
            pltpu.SemaphoreType.DMA,
            pltpu.SemaphoreType.DMA,
            pltpu.SemaphoreType.DMA,
            pltpu.SemaphoreType.DMA,
        ],
    )(_sc_body)


BT = 256


def _tc_body(cidf_ref, qid_ref, qbid_ref, lid_ref, qm_ref, lmeta_ref,
             qp_ref, qt_ref, lp_ref, lt_ref, ltype_ref,
             W1_ref, b1_ref, W2_ref, b2_ref, g_ref, beta_ref, out_ref):
    f32 = jnp.float32
    cidf = cidf_ref[...]
    is_q = (cidf < float(N_Q)).astype(f32)
    qm = qm_ref[...]
    lidx_f = jnp.maximum(cidf - float(N_Q), 0.0)
    iota_nl = lax.broadcasted_iota(jnp.int32, (1, N_L), 1).astype(f32)
    l_oh = (lidx_f == iota_nl).astype(f32)
    lm = jnp.dot(l_oh, lmeta_ref[...], preferred_element_type=f32)
    iota8 = lax.broadcasted_iota(jnp.int32, (1, 8), 1)
    iota189 = lax.broadcasted_iota(jnp.int32, (1, 189), 1)
    iota4 = lax.broadcasted_iota(jnp.int32, (1, 4), 1)
    iota8f = iota8.astype(f32)
    iota189f = iota189.astype(f32)
    iota4f = iota4.astype(f32)

    qp_oh = (qm[:, 1:2] == iota8).astype(f32)
    qp_e = jnp.dot(qp_oh, qp_ref[...], preferred_element_type=f32)

    M = jnp.zeros((BT, 189), f32)
    denom = jnp.zeros((BT, 1), f32)
    for j in range(6):
        tj = qm[:, 2 + j:3 + j]
        mj = (tj > 0).astype(f32)
        M = M + (tj == iota189).astype(f32) * mj
        denom = denom + mj
    tag_sum = jnp.dot(M, qt_ref[...], preferred_element_type=f32)
    tag_e = jnp.where(denom > 0, tag_sum / jnp.maximum(denom, 1e-9), 0.0)

    lp_oh = (lm[:, 0:1] == iota8f).astype(f32)
    lp_e = jnp.dot(lp_oh, lp_ref[...], preferred_element_type=f32)
    lt_oh = (lm[:, 1:2] == iota189f).astype(f32)
    lt_e = jnp.dot(lt_oh, lt_ref[...], preferred_element_type=f32)
    ltype_oh = (lm[:, 2:3] == iota4f).astype(f32)
    ltype_e = jnp.dot(ltype_oh, ltype_ref[...], preferred_element_type=f32)

    q_cat = jnp.concatenate([qid_ref[...], qbid_ref[...], qp_e, tag_e], axis=1) * is_q
    l_cat = jnp.concatenate([lid_ref[...], lp_e, lt_e, ltype_e], axis=1) * (1.0 - is_q)
    out = (jnp.dot(q_cat, W1_ref[...], preferred_element_type=f32)
           + jnp.dot(l_cat, W2_ref[...], preferred_element_type=f32))
    out = out + is_q * b1_ref[...] + (1.0 - is_q) * b2_ref[...]
    mu = jnp.mean(out, axis=1, keepdims=True)
    var = jnp.mean(jnp.square(out - mu), axis=1, keepdims=True)
    out_ref[...] = (out - mu) / jnp.sqrt(var + 1e-6) * g_ref[...] + beta_ref[...]


def _tok_spec(cols):
    return pl.BlockSpec((BT, cols), lambda i: (i, 0))


def _full_spec(shape):
    return pl.BlockSpec(shape, lambda i: (0,) * len(shape))


_tc_compute = pl.pallas_call(
    _tc_body,
    grid=(TS // BT,),
    in_specs=[
        _tok_spec(1),
        _tok_spec(D), _tok_spec(D), _tok_spec(D),
        _tok_spec(8), _full_spec((N_L, 8)),
        _full_spec((8, D)), _full_spec((189, D)),
        _full_spec((8, D)), _full_spec((189, D)), _full_spec((4, D)),
        _full_spec((4 * D, 4 * D)), _full_spec((1, 4 * D)),
        _full_spec((4 * D, 4 * D)), _full_spec((1, 4 * D)),
        _full_spec((1, 4 * D)), _full_spec((1, 4 * D)),
    ],
    out_specs=_tok_spec(4 * D),
    out_shape=jax.ShapeDtypeStruct((TS, 4 * D), jnp.float32),
)


def kernel(content_id, q_bundle, q_part, q_tags, l_part, l_tag, l_type,
           qid_table, qbid_table, qp_table, qt_table, lid_table, lp_table,
           lt_table, ltype_table, W1, b1, W2, b2, ln_gamma, ln_beta):
    B, S = content_id.shape
    cid = content_id.reshape(B * S).astype(jnp.int32)
    qmeta = jnp.concatenate(
        [q_bundle[:, None], q_part[:, None], q_tags], axis=1).astype(jnp.int32)
    lmeta_f = jnp.concatenate(
        [l_part[:, None], l_tag[:, None], l_type[:, None],
         jnp.zeros((N_L, 5), jnp.int32)], axis=1).astype(jnp.float32)

    qbun = q_bundle.astype(jnp.int32)
    sc = _get_sc_gather()
    outs = []
    for k in range(NSLICE):
        cid_k = lax.dynamic_slice_in_dim(cid, k * TS, TS)
        qid_e, qbid_e, lid_e, qm = sc(
            cid_k, qbun, qmeta, qid_table, qbid_table, lid_table)
        cidf = cid_k.reshape(TS, 1).astype(jnp.float32)
        outs.append(_tc_compute(
            cidf, qid_e, qbid_e, lid_e, qm, lmeta_f,
            qp_table, qt_table, lp_table, lt_table, ltype_table,
            W1, b1.reshape(1, -1), W2, b2.reshape(1, -1),
            ln_gamma.reshape(1, -1), ln_beta.reshape(1, -1)))
    out = jnp.concatenate(outs, axis=0)
    return out.reshape(B, S, 4 * D)


# speedup vs baseline: 1.5184x; 1.5184x over previous
import functools

import jax
import jax.numpy as jnp
from jax import lax
from jax.experimental import pallas as pl
from jax.experimental.pallas import tpu as pltpu
from jax.experimental.pallas import tpu_sc as plsc

N_Q = 100000
N_L = 1000
D = 64
T = 1024 * 200

NC = 2
NS = 16
LANES = 16
NW = NC * NS
NSLICE = 5
TS = T // NSLICE
TPW = TS // NW
CHUNK = 128
NPAIR = TPW // (2 * CHUNK)


def _sc_body(cid_hbm, qbun_hbm, qmeta_hbm, qid_t, qbid_t, lid_t,
             qid_o, qbid_o, lid_o, qm_o,
             cid_v, qidx_v, lidx_v, bund_v, qid_v, qbid_v, lid_v, qm_v,
             semg_a, semg_b, semw_a, semw_b):
    wid = lax.axis_index("s") * NC + lax.axis_index("c")
    wbase = wid * TPW
    semg = (semg_a, semg_b)
    semw = (semw_a, semw_b)

    def load_idx(h, base):
        pltpu.sync_copy(cid_hbm.at[pl.ds(base, CHUNK)], cid_v.at[h])
        for j in range(CHUNK // LANES):
            sl = pl.ds(j * LANES, LANES)
            cv = cid_v[h, sl]
            qidx_v[h, sl] = jnp.minimum(cv, N_Q - 1)
            lidx_v[h, sl] = jnp.maximum(cv - N_Q, 0)

    def fire_gathers(h):
        qix = qidx_v.at[h]
        lix = lidx_v.at[h]
        return [
            pltpu.async_copy(qbun_hbm.at[qix], bund_v.at[h], semg[h]),
            pltpu.async_copy(qmeta_hbm.at[qix], qm_v.at[h], semg[h]),
            pltpu.async_copy(qid_t.at[qix], qid_v.at[h], semg[h]),
            pltpu.async_copy(lid_t.at[lix], lid_v.at[h], semg[h]),
        ]

    def pair_body(p, carry):
        base_a = wbase + p * (2 * CHUNK)
        base_b = base_a + CHUNK
        load_idx(0, base_a)
        g_a = fire_gathers(0)
        load_idx(1, base_b)
        g_b = fire_gathers(1)
        for d in g_a:
            d.wait()
        h2_a = pltpu.async_copy(qbid_t.at[bund_v.at[0]], qbid_v.at[0], semg[0])
        w_a = [
            pltpu.async_copy(qid_v.at[0], qid_o.at[pl.ds(base_a, CHUNK)], semw[0]),
            pltpu.async_copy(lid_v.at[0], lid_o.at[pl.ds(base_a, CHUNK)], semw[0]),
            pltpu.async_copy(qm_v.at[0], qm_o.at[pl.ds(base_a, CHUNK)], semw[0]),
        ]
        for d in g_b:
            d.wait()
        h2_b = pltpu.async_copy(qbid_t.at[bund_v.at[1]], qbid_v.at[1], semg[1])
        w_b = [
            pltpu.async_copy(qid_v.at[1], qid_o.at[pl.ds(base_b, CHUNK)], semw[1]),
            pltpu.async_copy(lid_v.at[1], lid_o.at[pl.ds(base_b, CHUNK)], semw[1]),
            pltpu.async_copy(qm_v.at[1], qm_o.at[pl.ds(base_b, CHUNK)], semw[1]),
        ]
        h2_a.wait()
        w_a.append(pltpu.async_copy(qbid_v.at[0], qbid_o.at[pl.ds(base_a, CHUNK)], semw[0]))
        h2_b.wait()
        w_b.append(pltpu.async_copy(qbid_v.at[1], qbid_o.at[pl.ds(base_b, CHUNK)], semw[1]))
        for d in w_a:
            d.wait()
        for d in w_b:
            d.wait()
        return carry

    lax.fori_loop(0, NPAIR, pair_body, 0)


@functools.lru_cache(maxsize=1)
def _get_sc_gather():
    return functools.partial(
        pl.kernel,
        mesh=plsc.VectorSubcoreMesh(core_axis_name="c", subcore_axis_name="s"),
        compiler_params=pltpu.CompilerParams(use_tc_tiling_on_sc=False),
        out_type=[
            jax.ShapeDtypeStruct((TS, D), jnp.bfloat16),
            jax.ShapeDtypeStruct((TS, D), jnp.bfloat16),
            jax.ShapeDtypeStruct((TS, D), jnp.bfloat16),
            jax.ShapeDtypeStruct((TS, 8), jnp.int32),
        ],
        scratch_types=[
            pltpu.VMEM((2, CHUNK), jnp.int32),
            pltpu.VMEM((2, CHUNK), jnp.int32),
            pltpu.VMEM((2, CHUNK), jnp.int32),
            pltpu.VMEM((2, CHUNK), jnp.int32),
            pltpu.VMEM((2, CHUNK, D), jnp.bfloat16),
            pltpu.VMEM((2, CHUNK, D), jnp.bfloat16),
            pltpu.VMEM((2, CHUNK, D), jnp.bfloat16),
            pltpu.VMEM((2, CHUNK, 8), jnp.int32),
            pltpu.SemaphoreType.DMA,
            pltpu.SemaphoreType.DMA,
            pltpu.SemaphoreType.DMA,
            pltpu.SemaphoreType.DMA,
        ],
    )(_sc_body)


BT = 256


def _tc_body(cidf_ref, qid_ref, qbid_ref, lid_ref, qm_ref, lmeta_ref,
             qp_ref, qt_ref, lp_ref, lt_ref, ltype_ref,
             W1_ref, b1_ref, W2_ref, b2_ref, g_ref, beta_ref, out_ref):
    f32 = jnp.float32
    cidf = cidf_ref[...]
    is_q = (cidf < float(N_Q)).astype(f32)
    qm = qm_ref[...]
    lidx_f = jnp.maximum(cidf - float(N_Q), 0.0)
    iota_nl = lax.broadcasted_iota(jnp.int32, (1, N_L), 1).astype(f32)
    l_oh = (lidx_f == iota_nl).astype(f32)
    lm = jnp.dot(l_oh, lmeta_ref[...], preferred_element_type=f32)
    iota8 = lax.broadcasted_iota(jnp.int32, (1, 8), 1)
    iota189 = lax.broadcasted_iota(jnp.int32, (1, 189), 1)
    iota4 = lax.broadcasted_iota(jnp.int32, (1, 4), 1)
    iota8f = iota8.astype(f32)
    iota189f = iota189.astype(f32)
    iota4f = iota4.astype(f32)

    qp_oh = (qm[:, 1:2] == iota8).astype(f32)
    qp_e = jnp.dot(qp_oh, qp_ref[...], preferred_element_type=f32)

    M = jnp.zeros((BT, 189), f32)
    denom = jnp.zeros((BT, 1), f32)
    for j in range(6):
        tj = qm[:, 2 + j:3 + j]
        mj = (tj > 0).astype(f32)
        M = M + (tj == iota189).astype(f32) * mj
        denom = denom + mj
    tag_sum = jnp.dot(M, qt_ref[...], preferred_element_type=f32)
    tag_e = jnp.where(denom > 0, tag_sum / jnp.maximum(denom, 1e-9), 0.0)

    lp_oh = (lm[:, 0:1] == iota8f).astype(f32)
    lp_e = jnp.dot(lp_oh, lp_ref[...], preferred_element_type=f32)
    lt_oh = (lm[:, 1:2] == iota189f).astype(f32)
    lt_e = jnp.dot(lt_oh, lt_ref[...], preferred_element_type=f32)
    ltype_oh = (lm[:, 2:3] == iota4f).astype(f32)
    ltype_e = jnp.dot(ltype_oh, ltype_ref[...], preferred_element_type=f32)

    q_cat = jnp.concatenate(
        [qid_ref[...].astype(f32), qbid_ref[...].astype(f32), qp_e, tag_e],
        axis=1) * is_q
    l_cat = jnp.concatenate(
        [lid_ref[...].astype(f32), lp_e, lt_e, ltype_e], axis=1) * (1.0 - is_q)
    out = (jnp.dot(q_cat, W1_ref[...], preferred_element_type=f32)
           + jnp.dot(l_cat, W2_ref[...], preferred_element_type=f32))
    out = out + is_q * b1_ref[...] + (1.0 - is_q) * b2_ref[...]
    mu = jnp.mean(out, axis=1, keepdims=True)
    var = jnp.mean(jnp.square(out - mu), axis=1, keepdims=True)
    out_ref[...] = (out - mu) / jnp.sqrt(var + 1e-6) * g_ref[...] + beta_ref[...]


def _tok_spec(cols):
    return pl.BlockSpec((BT, cols), lambda i: (i, 0))


def _full_spec(shape):
    return pl.BlockSpec(shape, lambda i: (0,) * len(shape))


_tc_compute = pl.pallas_call(
    _tc_body,
    grid=(TS // BT,),
    in_specs=[
        _tok_spec(1),
        _tok_spec(D), _tok_spec(D), _tok_spec(D),
        _tok_spec(8), _full_spec((N_L, 8)),
        _full_spec((8, D)), _full_spec((189, D)),
        _full_spec((8, D)), _full_spec((189, D)), _full_spec((4, D)),
        _full_spec((4 * D, 4 * D)), _full_spec((1, 4 * D)),
        _full_spec((4 * D, 4 * D)), _full_spec((1, 4 * D)),
        _full_spec((1, 4 * D)), _full_spec((1, 4 * D)),
    ],
    out_specs=_tok_spec(4 * D),
    out_shape=jax.ShapeDtypeStruct((TS, 4 * D), jnp.float32),
)


def kernel(content_id, q_bundle, q_part, q_tags, l_part, l_tag, l_type,
           qid_table, qbid_table, qp_table, qt_table, lid_table, lp_table,
           lt_table, ltype_table, W1, b1, W2, b2, ln_gamma, ln_beta):
    B, S = content_id.shape
    cid = content_id.reshape(B * S).astype(jnp.int32)
    qmeta = jnp.concatenate(
        [q_bundle[:, None], q_part[:, None], q_tags], axis=1).astype(jnp.int32)
    lmeta_f = jnp.concatenate(
        [l_part[:, None], l_tag[:, None], l_type[:, None],
         jnp.zeros((N_L, 5), jnp.int32)], axis=1).astype(jnp.float32)

    qbun = q_bundle.astype(jnp.int32)
    sc = _get_sc_gather()
    outs = []
    for k in range(NSLICE):
        cid_k = lax.dynamic_slice_in_dim(cid, k * TS, TS)
        qid_e, qbid_e, lid_e, qm = sc(
            cid_k, qbun, qmeta, qid_table.astype(jnp.bfloat16),
            qbid_table.astype(jnp.bfloat16), lid_table.astype(jnp.bfloat16))
        cidf = cid_k.reshape(TS, 1).astype(jnp.float32)
        outs.append(_tc_compute(
            cidf, qid_e, qbid_e, lid_e, qm, lmeta_f,
            qp_table, qt_table, lp_table, lt_table, ltype_table,
            W1, b1.reshape(1, -1), W2, b2.reshape(1, -1),
            ln_gamma.reshape(1, -1), ln_beta.reshape(1, -1)))
    out = jnp.concatenate(outs, axis=0)
    return out.reshape(B, S, 4 * D)
